# baseline (speedup 1.0000x reference)
import jax
import jax.numpy as jnp
from jax import lax
from jax.experimental import pallas as pl
from jax.experimental.pallas import tpu as pltpu
from jax.experimental.pallas import tpu_sc as plsc

N = 100000
E = 1600000
H = 64
HP = 128
NC = 2
NS = 16
NW = NC * NS
NBINS = 16
RB = 6400
NP = NBINS * RB
BPC = NBINS // NC
LCAP = 320
WCAP = 16 * LCAP
BINSZ = NW * WCAP
TOT = NBINS * BINSZ
RPW = WCAP // 128
ROWS_Q = (E // 128) // NW
ROWS_REM = (E // 128) % NW
CA = 256
RPT = RB // NS
DPT = RB // NS


def _mesh():
    return plsc.VectorSubcoreMesh(core_axis_name="c", subcore_axis_name="s")


def _bin_body(src_h, dst_h, w_h, bsrc_h, bdst_h, bw_h,
              in_s, in_d, in_w, posr, zb_i, zb_f, curv):
    c = lax.axis_index("c")
    s = lax.axis_index("s")
    wid = s * NC + c

    zi = jnp.zeros((16,), jnp.int32)
    zf = jnp.zeros((16,), jnp.float32)

    def zv(g, _):
        zb_i[pl.ds(g * 16, 16)] = zi
        zb_f[pl.ds(g * 16, 16)] = zf
        return 0
    lax.fori_loop(0, WCAP // 16, zv, 0)
    for b in range(NBINS):
        off = b * BINSZ + wid * WCAP
        pltpu.sync_copy(zb_i, bsrc_h.at[pl.ds(off, WCAP)])
        pltpu.sync_copy(zb_i, bdst_h.at[pl.ds(off, WCAP)])
        pltpu.sync_copy(zb_f, bw_h.at[pl.ds(off, WCAP)])
    for b in range(NBINS):
        curv[pl.ds(b * 16, 16)] = zi

    nrows = ROWS_Q + jnp.minimum(jnp.maximum(ROWS_REM - wid, 0), 1)
    r0 = wid * ROWS_Q + jnp.minimum(wid, ROWS_REM)

    lanebase = lax.iota(jnp.int32, 16) * jnp.full((16,), LCAP, jnp.int32)
    capv = jnp.full((16,), LCAP - 1, jnp.int32)

    def row(r, carry):
        base = r * 128
        pltpu.sync_copy(src_h.at[pl.ds(base, 128)], in_s)
        pltpu.sync_copy(dst_h.at[pl.ds(base, 128)], in_d)
        pltpu.sync_copy(w_h.at[pl.ds(base, 128)], in_w)
        for q in range(8):
            d16 = in_d[pl.ds(q * 16, 16)]
            b16 = lax.div(d16, jnp.full((16,), RB, jnp.int32))
            dl16 = d16 - b16 * jnp.full((16,), RB, jnp.int32)
            in_d[pl.ds(q * 16, 16)] = dl16
            P = jnp.zeros((16,), jnp.int32)
            for b in range(NBINS):
                df = b16 - jnp.full((16,), b, jnp.int32)
                mi = jnp.full((16,), 1, jnp.int32) - jnp.minimum(
                    df * df, jnp.full((16,), 1, jnp.int32))
                cnt = curv[pl.ds(b * 16, 16)]
                off = jnp.full((16,), b * BINSZ + wid * WCAP, jnp.int32)
                idx = off + lanebase + jnp.minimum(cnt, capv)
                P = P + mi * idx
                curv[pl.ds(b * 16, 16)] = cnt + mi
            posr[0, pl.ds(q * 16, 16)] = P
        pltpu.sync_copy(in_s, bsrc_h.at[posr.at[0]])
        pltpu.sync_copy(in_d, bdst_h.at[posr.at[0]])
        pltpu.sync_copy(in_w, bw_h.at[posr.at[0]])
        return carry

    lax.fori_loop(r0, r0 + nrows, row, jnp.int32(0))


def _bin_call(src, dst, w):
    return pl.kernel(
        _bin_body,
        out_type=[
            jax.ShapeDtypeStruct((TOT,), jnp.int32),
            jax.ShapeDtypeStruct((TOT,), jnp.int32),
            jax.ShapeDtypeStruct((TOT,), jnp.float32),
        ],
        mesh=_mesh(),
        scratch_types=[
            pltpu.VMEM((128,), jnp.int32),
            pltpu.VMEM((128,), jnp.int32),
            pltpu.VMEM((128,), jnp.float32),
            pltpu.VMEM((1, 128), jnp.int32),
            pltpu.VMEM((WCAP,), jnp.int32),
            pltpu.VMEM((WCAP,), jnp.float32),
            pltpu.VMEM((NBINS * 16,), jnp.int32),
        ],
    )(src, dst, w)


def _deg_body(bdst_h, bw_h, deg_h, dstv, wv, zb,
              acc_0, acc_1, acc_2, acc_3, acc_4, acc_5, acc_6, acc_7):
    c = lax.axis_index("c")
    s = lax.axis_index("s")
    accs = (acc_0, acc_1, acc_2, acc_3, acc_4, acc_5, acc_6, acc_7)

    zf = jnp.zeros((16,), jnp.float32)

    def zz(g, _):
        zb[pl.ds(g * 16, 16)] = zf
        return 0
    lax.fori_loop(0, DPT // 16, zz, 0)
    for bl in range(BPC):
        pltpu.sync_copy(zb, accs[bl].at[pl.ds(s * DPT, DPT)])
    plsc.subcore_barrier()

    for bl in range(BPC):
        b = c * BPC + bl
        acc = accs[bl]
        for s2 in range(2):
            seg = s * 2 + s2
            row0 = b * (BINSZ // 128) + seg * RPW
            off = b * BINSZ + seg * WCAP
            pltpu.sync_copy(bdst_h.at[pl.ds(row0, RPW)], dstv)
            pltpu.sync_copy(bw_h.at[pl.ds(off, WCAP)], wv)

            def sadd(j, _):
                pltpu.sync_copy(wv.at[pl.ds(j * 128, 128)],
                                acc.at[dstv.at[j]], add=True)
                return 0
            lax.fori_loop(0, RPW, sadd, 0)

    plsc.subcore_barrier()
    for bl in range(BPC):
        b = c * BPC + bl
        pltpu.sync_copy(accs[bl].at[pl.ds(s * DPT, DPT)], zb)
        pltpu.sync_copy(zb, deg_h.at[pl.ds(b * RB + s * DPT, DPT)])


def _deg_call(bdst, bw):
    return pl.kernel(
        _deg_body,
        out_type=jax.ShapeDtypeStruct((NP,), jnp.float32),
        mesh=_mesh(),
        scratch_types=[
            pltpu.VMEM((RPW, 128), jnp.int32),
            pltpu.VMEM((WCAP,), jnp.float32),
            pltpu.VMEM((DPT,), jnp.float32),
        ] + [pltpu.VMEM_SHARED((RB,), jnp.float32) for _ in range(BPC)],
    )(bdst, bw)


def _agg_body(y_h, bsrc_h, bdst_h, bw_h, agg_h,
              srcv, dstv, wv, rows, zbuf, acc, sem):
    c = lax.axis_index("c")
    s = lax.axis_index("s")

    zf = jnp.zeros((16,), jnp.float32)

    def zz(r, _):
        for q in range(HP // 16):
            zbuf[r, pl.ds(q * 16, 16)] = zf
        return 0
    lax.fori_loop(0, 200, zz, 0)

    for bl in range(BPC):
        b = c * BPC + bl
        for k8 in range(RPT // 200):
            pltpu.sync_copy(zbuf, acc.at[pl.ds(s * RPT + k8 * 200, 200)])
        plsc.subcore_barrier()

        for s2 in range(2):
            seg = s * 2 + s2

            def chunk(ch, _):
                off = b * BINSZ + seg * WCAP + ch * CA
                row0 = b * (BINSZ // 128) + seg * RPW + ch * (CA // 128)
                pltpu.sync_copy(bsrc_h.at[pl.ds(off, CA)], srcv)
                pltpu.sync_copy(bdst_h.at[pl.ds(row0, CA // 128)], dstv)
                pltpu.sync_copy(bw_h.at[pl.ds(off, CA)], wv)

                hs = [pltpu.async_copy(
                        y_h.at[srcv.at[pl.ds(g * 128, 128)]],
                        rows.at[pl.ds(g * 128, 128)], sem)
                      for g in range(CA // 128)]
                for h in hs:
                    h.wait()

                def grp(g, _):
                    w16 = wv[pl.ds(g * 16, 16)]
                    for j in range(16):
                        e = g * 16 + j
                        wb = jnp.full((16,), w16[j], jnp.float32)
                        for q in range(H // 16):
                            rows[e, pl.ds(q * 16, 16)] = (
                                rows[e, pl.ds(q * 16, 16)] * wb)
                    return 0
                lax.fori_loop(0, CA // 16, grp, 0)

                for g in range(CA // 128):
                    pltpu.sync_copy(rows.at[pl.ds(g * 128, 128)],
                                    acc.at[dstv.at[g]], add=True)
                return 0
            lax.fori_loop(0, WCAP // CA, chunk, 0)

        plsc.subcore_barrier()
        for k8 in range(RPT // 200):
            pltpu.sync_copy(acc.at[pl.ds(s * RPT + k8 * 200, 200)], zbuf)
            pltpu.sync_copy(
                zbuf, agg_h.at[pl.ds(b * RB + s * RPT + k8 * 200, 200)])
        plsc.subcore_barrier()
        lax.fori_loop(0, 200, zz, 0)


def _agg_call(y, bsrc, bdst, bw):
    return pl.kernel(
        _agg_body,
        out_type=jax.ShapeDtypeStruct((NP, HP), jnp.float32),
        mesh=_mesh(),
        scratch_types=[
            pltpu.VMEM((CA,), jnp.int32),
            pltpu.VMEM((CA // 128, 128), jnp.int32),
            pltpu.VMEM((CA,), jnp.float32),
            pltpu.VMEM((CA, HP), jnp.float32),
            pltpu.VMEM((200, HP), jnp.float32),
            pltpu.VMEM_SHARED((RB, HP), jnp.float32),
            pltpu.SemaphoreType.DMA,
        ],
    )(y, bsrc, bdst, bw)


_BM = 1024


def _l1_body(x_ref, deg_ref, w_ref, y_ref, dinv_ref):
    dinv = lax.rsqrt(deg_ref[...] + 1.0)
    dinv_ref[...] = dinv
    y_ref[...] = (
        jnp.dot(x_ref[...], w_ref[...], preferred_element_type=jnp.float32)
        * dinv
    )


def _l1_call(xp, deg, W1p):
    return pl.pallas_call(
        _l1_body,
        grid=(NP // _BM,),
        in_specs=[
            pl.BlockSpec((_BM, 8), lambda i: (i, 0)),
            pl.BlockSpec((_BM, 1), lambda i: (i, 0)),
            pl.BlockSpec((8, HP), lambda i: (0, 0)),
        ],
        out_specs=[
            pl.BlockSpec((_BM, HP), lambda i: (i, 0)),
            pl.BlockSpec((_BM, 1), lambda i: (i, 0)),
        ],
        out_shape=[
            jax.ShapeDtypeStruct((NP, HP), jnp.float32),
            jax.ShapeDtypeStruct((NP, 1), jnp.float32),
        ],
    )(xp, deg, W1p)


def _lk_body(agg_ref, y_ref, dinv_ref, b_ref, w_ref, out_ref):
    h = jnp.maximum(
        dinv_ref[...] * (agg_ref[...] + y_ref[...]) + b_ref[...], 0.0)
    out_ref[...] = (
        jnp.dot(h, w_ref[...], preferred_element_type=jnp.float32)
        * dinv_ref[...]
    )


def _lk_call(agg, y, dinv, bk, Wk):
    return pl.pallas_call(
        _lk_body,
        grid=(NP // _BM,),
        in_specs=[
            pl.BlockSpec((_BM, HP), lambda i: (i, 0)),
            pl.BlockSpec((_BM, HP), lambda i: (i, 0)),
            pl.BlockSpec((_BM, 1), lambda i: (i, 0)),
            pl.BlockSpec((1, HP), lambda i: (0, 0)),
            pl.BlockSpec((HP, HP), lambda i: (0, 0)),
        ],
        out_specs=pl.BlockSpec((_BM, HP), lambda i: (i, 0)),
        out_shape=jax.ShapeDtypeStruct((NP, HP), jnp.float32),
    )(agg, y, dinv, bk, Wk)


def _fin_body(agg_ref, y_ref, dinv_ref, b_ref, wr_ref, br_ref, out_ref, m_ref):
    i = pl.program_id(0)
    h = jnp.maximum(
        dinv_ref[...] * (agg_ref[...] + y_ref[...]) + b_ref[...], 0.0)
    m = jnp.max(h, axis=0, keepdims=True)

    @pl.when(i == 0)
    def _():
        m_ref[...] = m

    @pl.when(i > 0)
    def _():
        m_ref[...] = jnp.maximum(m_ref[...], m)

    @pl.when(i == NP // _BM - 1)
    def _():
        out_ref[...] = (
            jnp.dot(m_ref[...], wr_ref[...],
                    preferred_element_type=jnp.float32) + br_ref[...]
        )


def _fin_call(agg, y, dinv, b4, Wr, br):
    return pl.pallas_call(
        _fin_body,
        grid=(NP // _BM,),
        in_specs=[
            pl.BlockSpec((_BM, HP), lambda i: (i, 0)),
            pl.BlockSpec((_BM, HP), lambda i: (i, 0)),
            pl.BlockSpec((_BM, 1), lambda i: (i, 0)),
            pl.BlockSpec((1, HP), lambda i: (0, 0)),
            pl.BlockSpec((HP, 1), lambda i: (0, 0)),
            pl.BlockSpec((1, 1), lambda i: (0, 0)),
        ],
        out_specs=pl.BlockSpec((1, 1), lambda i: (0, 0)),
        out_shape=jax.ShapeDtypeStruct((1, 1), jnp.float32),
        scratch_shapes=[pltpu.VMEM((1, HP), jnp.float32)],
    )(agg, y, dinv, b4, Wr, br)


def kernel(vertex_features, edges, weights, W1, b1, W2, b2, W3, b3, W4, b4,
           Wr, br):
    src = edges[0]
    dst = edges[1]
    xp = jnp.pad(vertex_features, ((0, NP - N), (0, 2)))
    W1p = jnp.pad(W1, ((0, 2), (0, HP - H)))

    bsrc, bdst1, bw = _bin_call(src, dst, weights)
    bdst = bdst1.reshape(TOT // 128, 128)
    deg_flat = _deg_call(bdst, bw)
    deg = deg_flat.reshape(NP, 1)

    y, dinv = _l1_call(xp, deg, W1p)
    for (bk, Wk) in ((b1, W2), (b2, W3), (b3, W4)):
        agg = _agg_call(y, bsrc, bdst, bw)
        bkp = jnp.pad(bk.reshape(1, H), ((0, 0), (0, HP - H)))
        Wkp = jnp.pad(Wk, ((0, HP - H), (0, HP - H)))
        y = _lk_call(agg, y, dinv, bkp, Wkp)
    agg = _agg_call(y, bsrc, bdst, bw)
    b4p = jnp.pad(b4.reshape(1, H), ((0, 0), (0, HP - H)))
    Wrp = jnp.pad(Wr, ((0, HP - H), (0, 0)))
    out = _fin_call(agg, y, dinv, b4p, Wrp, br.reshape(1, 1))
    return jnp.squeeze(out)
